# baseline (speedup 1.0000x reference)
import functools

import jax
import jax.numpy as jnp
from jax import lax
from jax.experimental import pallas as pl
from jax.experimental.pallas import tpu as pltpu
from jax.experimental.pallas import tpu_sc as plsc

_NC = 2
_NS = 16
_NW = _NC * _NS
_BC = 2
_NBUF = 3


@functools.lru_cache(maxsize=None)
def _build(B, T, V, dtype_name):
    dtype = jnp.dtype(dtype_name)
    BPW = B // _NW
    NCH = BPW // _BC

    mesh = plsc.VectorSubcoreMesh(core_axis_name="c", subcore_axis_name="s")

    @functools.partial(
        pl.kernel,
        mesh=mesh,
        compiler_params=pltpu.CompilerParams(use_tc_tiling_on_sc=False),
        out_type=jax.ShapeDtypeStruct((B, T, V), dtype),
        scratch_types=[
            pltpu.VMEM((BPW, T), jnp.int32),
            [pltpu.VMEM((_BC, T, V), dtype) for _ in range(_NBUF)],
            [pltpu.SemaphoreType.DMA for _ in range(_NBUF)],
            [pltpu.SemaphoreType.DMA for _ in range(_NBUF)],
        ],
    )
    def bigram_gather(table_hbm, src_hbm, out_hbm, idx_v, bufs, gsems, ssems):
        wid = lax.axis_index("s") * _NC + lax.axis_index("c")
        base_b = wid * BPW

        pltpu.sync_copy(src_hbm.at[pl.ds(base_b, BPW)], idx_v)

        def gathers(c):
            k = c % _NBUF
            return [
                pltpu.make_async_copy(
                    table_hbm.at[idx_v.at[c * _BC + j]],
                    bufs[k].at[j], gsems[k])
                for j in range(_BC)
            ]

        def scatter(c):
            k = c % _NBUF
            return pltpu.make_async_copy(
                bufs[k],
                out_hbm.at[pl.ds(base_b + c * _BC, _BC)],
                ssems[k])

        for c in range(min(_NBUF, NCH)):
            for g in gathers(c):
                g.start()
        for c in range(NCH):
            for g in gathers(c):
                g.wait()
            scatter(c).start()
            if c + _NBUF < NCH:
                scatter(c).wait()
                for g in gathers(c + _NBUF):
                    g.start()
        for c in range(max(NCH - _NBUF, 0), NCH):
            scatter(c).wait()

    return bigram_gather


def kernel(idx, log_probs):
    B, T = idx.shape
    V = log_probs.shape[1]
    Vr = log_probs.shape[0]
    table = jnp.concatenate(
        [log_probs, jnp.zeros((1, V), log_probs.dtype)], axis=0)
    src = jnp.concatenate(
        [jnp.full((B, 1), Vr, jnp.int32), idx[:, :-1].astype(jnp.int32)],
        axis=1)
    return _build(B, T, V, log_probs.dtype.name)(table, src)


# speedup vs baseline: 1.4476x; 1.4476x over previous
import functools

import jax
import jax.numpy as jnp
from jax.experimental import pallas as pl
from jax.experimental.pallas import tpu as pltpu

_BB = 4


@functools.lru_cache(maxsize=None)
def _build_tc(B, T, V, Vr, dtype_name):
    dtype = jnp.dtype(dtype_name)

    def body(idx_ref, table_ref, out_ref):
        i = pl.program_id(0)
        zero = jnp.zeros((V,), dtype)
        for j in range(_BB):
            b = i * _BB + j
            out_ref[j, 0, :] = zero
            for t in range(1, T):
                row = idx_ref[b, t - 1]
                out_ref[j, t, :] = table_ref[row, :]

    return pl.pallas_call(
        body,
        grid_spec=pltpu.PrefetchScalarGridSpec(
            num_scalar_prefetch=1,
            grid=(B // _BB,),
            in_specs=[pl.BlockSpec((Vr, V), lambda i, *_: (0, 0))],
            out_specs=pl.BlockSpec((_BB, T, V), lambda i, *_: (i, 0, 0)),
        ),
        out_shape=jax.ShapeDtypeStruct((B, T, V), dtype),
    )


def kernel(idx, log_probs):
    B, T = idx.shape
    Vr, V = log_probs.shape
    return _build_tc(B, T, V, Vr, log_probs.dtype.name)(
        idx.astype(jnp.int32), log_probs)


# speedup vs baseline: 1.9857x; 1.3717x over previous
import functools

import jax
import jax.numpy as jnp
from jax.experimental import pallas as pl
from jax.experimental.pallas import tpu as pltpu

_BB = 16


@functools.lru_cache(maxsize=None)
def _build_tc(B, T, V, Vr, dtype_name):
    dtype = jnp.dtype(dtype_name)

    def body(idx_ref, table_ref, out_ref):
        i = pl.program_id(0)
        zero = jnp.zeros((V,), dtype)
        for j in range(_BB):
            b = i * _BB + j
            out_ref[j, 0, :] = zero
            for t in range(1, T):
                row = idx_ref[b, t - 1]
                out_ref[j, t, :] = table_ref[row, :]

    return pl.pallas_call(
        body,
        grid_spec=pltpu.PrefetchScalarGridSpec(
            num_scalar_prefetch=1,
            grid=(B // _BB,),
            in_specs=[pl.BlockSpec((Vr, V), lambda i, *_: (0, 0))],
            out_specs=pl.BlockSpec((_BB, T, V), lambda i, *_: (i, 0, 0)),
        ),
        out_shape=jax.ShapeDtypeStruct((B, T, V), dtype),
    )


def kernel(idx, log_probs):
    B, T = idx.shape
    Vr, V = log_probs.shape
    return _build_tc(B, T, V, Vr, log_probs.dtype.name)(
        idx.astype(jnp.int32), log_probs)
